# baseline (device time: 90042 ns/iter reference)
import jax
import jax.numpy as jnp
from jax import lax
from jax.experimental import pallas as pl
from jax.experimental.pallas import tpu as pltpu

N_DEV = 4


def kernel(x, router_W, route_idx, expert_W):
    n, d = x.shape
    n_local, _, h = expert_W.shape
    n_experts = router_W.shape[1]
    chunk = n // N_DEV

    def body(x_ref, rw_ref, idx_ref, ew_ref, out_ref,
             send_buf, recv_buf, send_sems, recv_sems):
        my = lax.axis_index("i")
        left = lax.rem(my + N_DEV - 1, N_DEV)
        right = lax.rem(my + 1, N_DEV)

        barrier_sem = pltpu.get_barrier_semaphore()
        for nbr in (left, right):
            pl.semaphore_signal(
                barrier_sem, inc=1,
                device_id=(nbr,), device_id_type=pl.DeviceIdType.MESH,
            )
        pl.semaphore_wait(barrier_sem, 2)

        def partial_chunk(c):
            r0 = c * chunk
            xs = x_ref[pl.ds(r0, chunk), :]
            idx = idx_ref[pl.ds(r0, chunk), :]
            scores = jnp.dot(xs, rw_ref[...],
                             preferred_element_type=jnp.float32)
            m = jnp.max(scores, axis=-1, keepdims=True)
            e = jnp.exp(scores - m)
            probs = e / jnp.sum(e, axis=-1, keepdims=True)
            iota = lax.broadcasted_iota(jnp.int32, (chunk, n_experts), 1)
            idx0 = idx[:, 0:1]
            idx1 = idx[:, 1:2]
            p0 = jnp.sum(probs * (iota == idx0), axis=-1, keepdims=True)
            p1 = jnp.sum(probs * (iota == idx1), axis=-1, keepdims=True)
            gs = p0 + p1
            g0 = p0 / gs
            g1 = p1 / gs
            acc = jnp.zeros((chunk, h), jnp.float32)
            for le in range(n_local):
                ge = my * n_local + le
                gate = (g0 * (idx0 == ge).astype(jnp.float32)
                        + g1 * (idx1 == ge).astype(jnp.float32))
                acc = acc + jnp.dot(xs * gate, ew_ref[le],
                                    preferred_element_type=jnp.float32)
            return acc

        c0 = lax.rem(my + N_DEV - 1, N_DEV)
        send_buf[...] = partial_chunk(c0)
        for hp in range(N_DEV - 1):
            rdma = pltpu.make_async_remote_copy(
                src_ref=send_buf,
                dst_ref=recv_buf.at[hp],
                send_sem=send_sems.at[hp],
                recv_sem=recv_sems.at[hp],
                device_id=(right,),
                device_id_type=pl.DeviceIdType.MESH,
            )
            rdma.start()
            c_next = lax.rem(my + 2 * N_DEV - 2 - hp, N_DEV)
            p = partial_chunk(c_next)
            rdma.wait()
            total = recv_buf[hp] + p
            if hp < N_DEV - 2:
                send_buf[...] = total
            else:
                out_ref[...] = total

    return pl.pallas_call(
        body,
        out_shape=jax.ShapeDtypeStruct((chunk, h), jnp.float32),
        in_specs=[
            pl.BlockSpec(memory_space=pltpu.VMEM),
            pl.BlockSpec(memory_space=pltpu.VMEM),
            pl.BlockSpec(memory_space=pltpu.VMEM),
            pl.BlockSpec(memory_space=pltpu.VMEM),
        ],
        out_specs=pl.BlockSpec(memory_space=pltpu.VMEM),
        scratch_shapes=[
            pltpu.VMEM((chunk, h), jnp.float32),
            pltpu.VMEM((N_DEV - 1, chunk, h), jnp.float32),
            pltpu.SemaphoreType.DMA((N_DEV - 1,)),
            pltpu.SemaphoreType.DMA((N_DEV - 1,)),
        ],
        compiler_params=pltpu.CompilerParams(collective_id=0),
    )(x, router_W, route_idx, expert_W)


# device time: 55103 ns/iter; 1.6341x vs baseline; 1.6341x over previous
import jax
import jax.numpy as jnp
from jax import lax
from jax.experimental import pallas as pl
from jax.experimental.pallas import tpu as pltpu

N_DEV = 4


def kernel(x, router_W, route_idx, expert_W):
    n, d = x.shape
    n_local, _, h = expert_W.shape
    n_experts = router_W.shape[1]
    chunk = n // N_DEV

    def body(x_ref, rw_ref, idx_ref, ew_ref, out_ref,
             ew_bf, send_buf, recv_buf, send_sems, recv_sems):
        my = lax.axis_index("i")
        left = lax.rem(my + N_DEV - 1, N_DEV)
        right = lax.rem(my + 1, N_DEV)

        barrier_sem = pltpu.get_barrier_semaphore()
        for nbr in (left, right):
            pl.semaphore_signal(
                barrier_sem, inc=1,
                device_id=(nbr,), device_id_type=pl.DeviceIdType.MESH,
            )
        pl.semaphore_wait(barrier_sem, 2)

        ew_bf[...] = ew_ref[...].astype(jnp.bfloat16)

        def partial_chunk(c):
            r0 = c * chunk
            xs = x_ref[pl.ds(r0, chunk), :]
            idx = idx_ref[pl.ds(r0, chunk), :]
            scores = jnp.dot(xs, rw_ref[...],
                             preferred_element_type=jnp.float32)
            m = jnp.max(scores, axis=-1, keepdims=True)
            e = jnp.exp(scores - m)
            probs = e / jnp.sum(e, axis=-1, keepdims=True)
            iota = lax.broadcasted_iota(jnp.int32, (chunk, n_experts), 1)
            idx0 = idx[:, 0:1]
            idx1 = idx[:, 1:2]
            p0 = jnp.sum(probs * (iota == idx0), axis=-1, keepdims=True)
            p1 = jnp.sum(probs * (iota == idx1), axis=-1, keepdims=True)
            gs = p0 + p1
            g0 = p0 / gs
            g1 = p1 / gs
            xs_bf = xs.astype(jnp.bfloat16)
            acc = jnp.zeros((chunk, h), jnp.float32)
            for le in range(n_local):
                ge = my * n_local + le
                gate = (g0 * (idx0 == ge).astype(jnp.float32)
                        + g1 * (idx1 == ge).astype(jnp.float32))
                acc = acc + gate * jnp.dot(xs_bf, ew_bf[le],
                                           preferred_element_type=jnp.float32)
            return acc

        c0 = lax.rem(my + N_DEV - 1, N_DEV)
        send_buf[...] = partial_chunk(c0).astype(jnp.bfloat16)
        for hp in range(N_DEV - 1):
            rdma = pltpu.make_async_remote_copy(
                src_ref=send_buf,
                dst_ref=recv_buf.at[hp],
                send_sem=send_sems.at[hp],
                recv_sem=recv_sems.at[hp],
                device_id=(right,),
                device_id_type=pl.DeviceIdType.MESH,
            )
            rdma.start()
            c_next = lax.rem(my + 2 * N_DEV - 2 - hp, N_DEV)
            p = partial_chunk(c_next)
            rdma.wait()
            total = recv_buf[hp].astype(jnp.float32) + p
            if hp < N_DEV - 2:
                send_buf[...] = total.astype(jnp.bfloat16)
            else:
                out_ref[...] = total

    return pl.pallas_call(
        body,
        out_shape=jax.ShapeDtypeStruct((chunk, h), jnp.float32),
        in_specs=[
            pl.BlockSpec(memory_space=pltpu.VMEM),
            pl.BlockSpec(memory_space=pltpu.VMEM),
            pl.BlockSpec(memory_space=pltpu.VMEM),
            pl.BlockSpec(memory_space=pltpu.VMEM),
        ],
        out_specs=pl.BlockSpec(memory_space=pltpu.VMEM),
        scratch_shapes=[
            pltpu.VMEM((n_local, d, h), jnp.bfloat16),
            pltpu.VMEM((chunk, h), jnp.bfloat16),
            pltpu.VMEM((N_DEV - 1, chunk, h), jnp.bfloat16),
            pltpu.SemaphoreType.DMA((N_DEV - 1,)),
            pltpu.SemaphoreType.DMA((N_DEV - 1,)),
        ],
        compiler_params=pltpu.CompilerParams(collective_id=0),
    )(x, router_W, route_idx, expert_W)


# device time: 38186 ns/iter; 2.3580x vs baseline; 1.4430x over previous
import jax
import jax.numpy as jnp
from jax import lax
from jax.experimental import pallas as pl
from jax.experimental.pallas import tpu as pltpu

N_DEV = 4


def kernel(x, router_W, route_idx, expert_W):
    n, d = x.shape
    n_local, _, h = expert_W.shape
    n_experts = router_W.shape[1]
    chunk = n // N_DEV
    hh = h // 2

    def body(x_ref, rw_ref, idx_ref, ew_ref, out_ref, ew_bf,
             cw_send, ccw_send, cw_recv, ccw_recv,
             cw_ssem, cw_rsem, ccw_ssem, ccw_rsem):
        my = lax.axis_index("i")
        left = lax.rem(my + N_DEV - 1, N_DEV)
        right = lax.rem(my + 1, N_DEV)

        barrier_sem = pltpu.get_barrier_semaphore()
        for nbr in (left, right):
            pl.semaphore_signal(
                barrier_sem, inc=1,
                device_id=(nbr,), device_id_type=pl.DeviceIdType.MESH,
            )
        pl.semaphore_wait(barrier_sem, 2)

        ew_bf[...] = ew_ref[...].astype(jnp.bfloat16)

        def partial_half(c, lo):
            r0 = c * chunk
            xs = x_ref[pl.ds(r0, chunk), :]
            idx = idx_ref[pl.ds(r0, chunk), :]
            scores = jnp.dot(xs, rw_ref[...],
                             preferred_element_type=jnp.float32)
            m = jnp.max(scores, axis=-1, keepdims=True)
            e = jnp.exp(scores - m)
            probs = e / jnp.sum(e, axis=-1, keepdims=True)
            iota = lax.broadcasted_iota(jnp.int32, (chunk, n_experts), 1)
            idx0 = idx[:, 0:1]
            idx1 = idx[:, 1:2]
            p0 = jnp.sum(probs * (iota == idx0), axis=-1, keepdims=True)
            p1 = jnp.sum(probs * (iota == idx1), axis=-1, keepdims=True)
            gs = p0 + p1
            g0 = p0 / gs
            g1 = p1 / gs
            xs_bf = xs.astype(jnp.bfloat16)
            acc = jnp.zeros((chunk, hh), jnp.float32)
            for le in range(n_local):
                ge = my * n_local + le
                gate = (g0 * (idx0 == ge).astype(jnp.float32)
                        + g1 * (idx1 == ge).astype(jnp.float32))
                acc = acc + gate * jnp.dot(xs_bf, ew_bf[le, :, lo:lo + hh],
                                           preferred_element_type=jnp.float32)
            return acc

        cw_send[...] = partial_half(lax.rem(my + N_DEV - 1, N_DEV),
                                    0).astype(jnp.bfloat16)
        ccw_send[...] = partial_half(lax.rem(my + 1, N_DEV),
                                     hh).astype(jnp.bfloat16)

        for hp in range(N_DEV - 1):
            cw = pltpu.make_async_remote_copy(
                src_ref=cw_send, dst_ref=cw_recv.at[hp],
                send_sem=cw_ssem.at[hp], recv_sem=cw_rsem.at[hp],
                device_id=(right,), device_id_type=pl.DeviceIdType.MESH,
            )
            ccw = pltpu.make_async_remote_copy(
                src_ref=ccw_send, dst_ref=ccw_recv.at[hp],
                send_sem=ccw_ssem.at[hp], recv_sem=ccw_rsem.at[hp],
                device_id=(left,), device_id_type=pl.DeviceIdType.MESH,
            )
            cw.start()
            ccw.start()
            c_cw = lax.rem(my + 2 * N_DEV - 2 - hp, N_DEV)
            c_ccw = lax.rem(my + 2 + hp, N_DEV)
            pa = partial_half(c_cw, 0)
            pb = partial_half(c_ccw, hh)
            cw.wait()
            ccw.wait()
            ta = cw_recv[hp].astype(jnp.float32) + pa
            tb = ccw_recv[hp].astype(jnp.float32) + pb
            if hp < N_DEV - 2:
                cw_send[...] = ta.astype(jnp.bfloat16)
                ccw_send[...] = tb.astype(jnp.bfloat16)
            else:
                out_ref[:, 0:hh] = ta
                out_ref[:, hh:h] = tb

    return pl.pallas_call(
        body,
        out_shape=jax.ShapeDtypeStruct((chunk, h), jnp.float32),
        in_specs=[
            pl.BlockSpec(memory_space=pltpu.VMEM),
            pl.BlockSpec(memory_space=pltpu.VMEM),
            pl.BlockSpec(memory_space=pltpu.VMEM),
            pl.BlockSpec(memory_space=pltpu.VMEM),
        ],
        out_specs=pl.BlockSpec(memory_space=pltpu.VMEM),
        scratch_shapes=[
            pltpu.VMEM((n_local, d, h), jnp.bfloat16),
            pltpu.VMEM((chunk, hh), jnp.bfloat16),
            pltpu.VMEM((chunk, hh), jnp.bfloat16),
            pltpu.VMEM((N_DEV - 1, chunk, hh), jnp.bfloat16),
            pltpu.VMEM((N_DEV - 1, chunk, hh), jnp.bfloat16),
            pltpu.SemaphoreType.DMA((N_DEV - 1,)),
            pltpu.SemaphoreType.DMA((N_DEV - 1,)),
            pltpu.SemaphoreType.DMA((N_DEV - 1,)),
            pltpu.SemaphoreType.DMA((N_DEV - 1,)),
        ],
        compiler_params=pltpu.CompilerParams(collective_id=0),
    )(x, router_W, route_idx, expert_W)
